# baseline (speedup 1.0000x reference)
import functools

import jax
import jax.numpy as jnp
import numpy as np
from jax import lax
from jax.experimental import pallas as pl
from jax.experimental.pallas import tpu as pltpu
from jax.experimental.pallas import tpu_sc as plsc

N = 10000
E = 320000
F_IN = 128
HID = 64
R = 8
NCLS = 32

NC = 2
NS = 16
NW = NC * NS
EPW = E // NW
CH = 80
JW = EPW // CH
ECT = E // NS
JC = ECT // CH
LANES = 16


def _norm_body(dst_hbm, et_hbm, norm_hbm,
               counts_sh, dst_v, et_v, cidx2, gcidx, cnt_v, ones_v, sem, sem2):
    cid = lax.axis_index("c")
    sid = lax.axis_index("s")
    wid = sid * NC + cid

    for k in range(CH // LANES):
        ones_v[pl.ds(k * LANES, LANES)] = jnp.ones((LANES,), jnp.float32)

    zch = (N * R) // NS

    def zero_iter(k, carry):
        cnt_v[pl.ds(k * LANES, LANES)] = jnp.zeros((LANES,), jnp.float32)
        return carry
    lax.fori_loop(0, EPW // LANES, zero_iter, None)
    pltpu.sync_copy(cnt_v.at[pl.ds(0, zch)],
                    counts_sh.at[pl.ds(sid * zch, zch)])
    plsc.subcore_barrier()

    pltpu.sync_copy(dst_hbm.at[pl.ds(sid * ECT, ECT)], dst_v)
    pltpu.sync_copy(et_hbm.at[pl.ds(sid * ECT, ECT)], et_v)

    def cidx_iter(j, carry):
        for k in range(CH // LANES):
            sl = pl.ds(j * CH + k * LANES, LANES)
            cidx2[j, pl.ds(k * LANES, LANES)] = dst_v[sl] * R + et_v[sl]
        return carry
    lax.fori_loop(0, JC, cidx_iter, None)

    kgrp = 10
    def cgrp(g, carry):
        for k in range(kgrp):
            pltpu.async_copy(ones_v, counts_sh.at[cidx2.at[g * kgrp + k]],
                             sem, add=True)
        for k in range(kgrp):
            pltpu.make_async_copy(ones_v, counts_sh.at[cidx2.at[g * kgrp + k]],
                                  sem).wait()
        return carry
    lax.fori_loop(0, JC // kgrp, cgrp, None)
    plsc.subcore_barrier()

    base = wid * EPW
    pltpu.sync_copy(dst_hbm.at[pl.ds(base, EPW)], dst_v.at[pl.ds(0, EPW)])
    pltpu.sync_copy(et_hbm.at[pl.ds(base, EPW)], et_v.at[pl.ds(0, EPW)])

    def gidx_iter(k, carry):
        sl = pl.ds(k * LANES, LANES)
        gcidx[sl] = dst_v[sl] * R + et_v[sl]
        return carry
    lax.fori_loop(0, EPW // LANES, gidx_iter, None)

    ggrp_n = 5
    def ggrp(g, carry):
        for k in range(ggrp_n):
            j = g * ggrp_n + k
            pltpu.async_copy(counts_sh.at[gcidx.at[pl.ds(j * CH, CH)]],
                             cnt_v.at[pl.ds(j * CH, CH)], sem2)
        for k in range(ggrp_n):
            j = g * ggrp_n + k
            pltpu.make_async_copy(counts_sh.at[gcidx.at[pl.ds(j * CH, CH)]],
                                  cnt_v.at[pl.ds(j * CH, CH)], sem2).wait()
        return carry
    lax.fori_loop(0, JW // ggrp_n, ggrp, None)

    def norm_iter(k, carry):
        sl = pl.ds(k * LANES, LANES)
        cnt_v[sl] = 1.0 / jnp.maximum(cnt_v[sl], 1.0)
        return carry
    lax.fori_loop(0, EPW // LANES, norm_iter, None)
    pltpu.sync_copy(cnt_v, norm_hbm.at[pl.ds(base, EPW)])


DEPTH = 4


def _agg_body(h_hbm, src_hbm, et_hbm, dst2_hbm, norm_hbm, part_hbm,
              acc_sh, src_v, et_v, dst2_v, norm_v,
              graw0, graw1, graw2, graw3, sbuf0, sbuf1, sbuf2, sbuf3,
              sg0, sg1, sg2, sg3, ss0, ss1, ss2, ss3):
    cid = lax.axis_index("c")
    sid = lax.axis_index("s")
    wid = sid * NC + cid

    NCHN = N // CH

    def zero_iter(r, carry):
        for c4 in range(HID // LANES):
            sbuf0[r, pl.ds(c4 * LANES, LANES)] = jnp.zeros((LANES,),
                                                           jnp.float32)
        return carry
    lax.fori_loop(0, CH, zero_iter, None)
    for kk in range((NCHN + NS - 1) // NS):
        cidk = sid + kk * NS
        @pl.when(cidk < NCHN)
        def _():
            pltpu.sync_copy(sbuf0, acc_sh.at[pl.ds(cidk * CH, CH)])
    plsc.subcore_barrier()

    base = wid * EPW
    pltpu.sync_copy(src_hbm.at[pl.ds(base, EPW)], src_v)
    pltpu.sync_copy(et_hbm.at[pl.ds(base, EPW)], et_v)
    pltpu.sync_copy(dst2_hbm.at[wid], dst2_v)
    pltpu.sync_copy(norm_hbm.at[pl.ds(base, EPW)], norm_v)

    def gidx_iter(k, carry):
        sl = pl.ds(k * LANES, LANES)
        src_v[sl] = src_v[sl] * R + et_v[sl]
        return carry
    lax.fori_loop(0, EPW // LANES, gidx_iter, None)

    graw = (graw0, graw1, graw2, graw3)
    sbuf = (sbuf0, sbuf1, sbuf2, sbuf3)
    sg = (sg0, sg1, sg2, sg3)
    ss = (ss0, ss1, ss2, ss3)

    def fire_g(j, b):
        pltpu.async_copy(h_hbm.at[src_v.at[pl.ds(j * CH, CH)]], graw[b], sg[b])

    def wait_g(j, b):
        pltpu.make_async_copy(h_hbm.at[src_v.at[pl.ds(j * CH, CH)]],
                              graw[b], sg[b]).wait()

    def fire_s(j, b):
        pltpu.async_copy(sbuf[b], acc_sh.at[dst2_v.at[j]], ss[b], add=True)

    def wait_s(j, b):
        pltpu.make_async_copy(sbuf[b], acc_sh.at[dst2_v.at[j]], ss[b]).wait()

    def scale(j, b):
        gb = graw[b]
        sb = sbuf[b]
        def sgrp(g, carry):
            nch = norm_v[pl.ds(j * CH + g * LANES, LANES)]
            for i in range(LANES):
                nv = nch[i]
                row = g * LANES + i
                for h2 in range(HID // (2 * LANES)):
                    w = gb[row, pl.ds(h2 * LANES, LANES)]
                    av = lax.bitcast_convert_type(w << 16, jnp.float32)
                    bv = lax.bitcast_convert_type(w & jnp.int32(-65536),
                                                  jnp.float32)
                    sb[row, pl.ds(h2 * 2 * LANES, LANES)] = av * nv
                    sb[row, pl.ds(h2 * 2 * LANES + LANES, LANES)] = bv * nv
            return carry
        lax.fori_loop(0, CH // LANES, sgrp, None)

    for jp in range(DEPTH - 1):
        fire_g(jp, jp)

    def quad(jq, carry):
        for b in range(DEPTH):
            j = jq * DEPTH + b
            @pl.when(j < JW)
            def _():
                @pl.when(j + DEPTH - 1 < JW)
                def _():
                    fire_g(j + DEPTH - 1, (b + DEPTH - 1) % DEPTH)
                wait_g(j, b)
                @pl.when(j >= DEPTH)
                def _():
                    wait_s(j - DEPTH, b)
                scale(j, b)
                fire_s(j, b)
        return carry
    lax.fori_loop(0, (JW + DEPTH - 1) // DEPTH, quad, None)
    for jt in range(JW - DEPTH, JW):
        wait_s(jt, jt % DEPTH)
    plsc.subcore_barrier()

    for kk in range((NCHN + NS - 1) // NS):
        cidk = sid + kk * NS
        @pl.when(cidk < NCHN)
        def _():
            pltpu.sync_copy(acc_sh.at[pl.ds(cidk * CH, CH)], sbuf0)
            pltpu.sync_copy(sbuf0, part_hbm.at[cid, pl.ds(cidk * CH, CH)])


def _sc_norm(dst, et):
    mesh = plsc.VectorSubcoreMesh(core_axis_name="c", subcore_axis_name="s")
    return pl.kernel(
        _norm_body,
        out_type=jax.ShapeDtypeStruct((E,), jnp.float32),
        mesh=mesh,
        compiler_params=pltpu.CompilerParams(use_tc_tiling_on_sc=False),
        scratch_types=[
            pltpu.VMEM_SHARED((N * R,), jnp.float32),
            pltpu.VMEM((ECT,), jnp.int32),
            pltpu.VMEM((ECT,), jnp.int32),
            pltpu.VMEM((JC, CH), jnp.int32),
            pltpu.VMEM((EPW,), jnp.int32),
            pltpu.VMEM((EPW,), jnp.float32),
            pltpu.VMEM((CH,), jnp.float32),
            pltpu.SemaphoreType.DMA,
            pltpu.SemaphoreType.DMA,
        ],
    )(dst, et)


def _sc_agg(h, src, et, dst2, norm):
    mesh = plsc.VectorSubcoreMesh(core_axis_name="c", subcore_axis_name="s")
    return pl.kernel(
        _agg_body,
        out_type=jax.ShapeDtypeStruct((NC, N, HID), jnp.float32),
        mesh=mesh,
        compiler_params=pltpu.CompilerParams(use_tc_tiling_on_sc=False),
        scratch_types=(
            [pltpu.VMEM_SHARED((N, HID), jnp.float32),
             pltpu.VMEM((EPW,), jnp.int32),
             pltpu.VMEM((EPW,), jnp.int32),
             pltpu.VMEM((JW, CH), jnp.int32),
             pltpu.VMEM((EPW,), jnp.float32)]
            + [pltpu.VMEM((CH, HID // 2), jnp.int32)] * DEPTH
            + [pltpu.VMEM((CH, HID), jnp.float32)] * DEPTH
            + [pltpu.SemaphoreType.DMA] * (2 * DEPTH)
        ),
    )(h, src, et, dst2, norm)


BR = 1000


def _t1_body(x_ref, w_ref, r_ref, b_ref, h_ref, s_ref):
    xb = x_ref[...]
    h_ref[...] = jnp.dot(
        xb, w_ref[...],
        preferred_element_type=jnp.float32).astype(jnp.bfloat16)
    s_ref[...] = (jnp.dot(xb, r_ref[...], preferred_element_type=jnp.float32)
                  + b_ref[...])


def _tc1(x, w1r, root1, b1):
    return pl.pallas_call(
        _t1_body,
        grid=(N // BR,),
        in_specs=[pl.BlockSpec((BR, F_IN), lambda i: (i, 0)),
                  pl.BlockSpec((F_IN, R * HID), lambda i: (0, 0)),
                  pl.BlockSpec((F_IN, HID), lambda i: (0, 0)),
                  pl.BlockSpec((1, HID), lambda i: (0, 0))],
        out_specs=[pl.BlockSpec((BR, R * HID), lambda i: (i, 0)),
                   pl.BlockSpec((BR, HID), lambda i: (i, 0))],
        out_shape=[jax.ShapeDtypeStruct((N, R * HID), jnp.bfloat16),
                   jax.ShapeDtypeStruct((N, HID), jnp.float32)],
    )(x, w1r, root1, b1)


def _t2_body(p_ref, s1_ref, w_ref, r_ref, b_ref, o1_ref, h2_ref, s2_ref):
    o1 = jnp.maximum(p_ref[0] + p_ref[1] + s1_ref[...], 0.0)
    o1_ref[...] = o1
    h2_ref[...] = jnp.dot(
        o1, w_ref[...],
        preferred_element_type=jnp.float32).astype(jnp.bfloat16)
    s2_ref[...] = (jnp.dot(o1, r_ref[...], preferred_element_type=jnp.float32)
                   + b_ref[...])


def _tc2(p1, s1, w2r, root2, b2):
    return pl.pallas_call(
        _t2_body,
        grid=(N // BR,),
        in_specs=[pl.BlockSpec((NC, BR, HID), lambda i: (0, i, 0)),
                  pl.BlockSpec((BR, HID), lambda i: (i, 0)),
                  pl.BlockSpec((HID, R * HID), lambda i: (0, 0)),
                  pl.BlockSpec((HID, HID), lambda i: (0, 0)),
                  pl.BlockSpec((1, HID), lambda i: (0, 0))],
        out_specs=[pl.BlockSpec((BR, HID), lambda i: (i, 0)),
                   pl.BlockSpec((BR, R * HID), lambda i: (i, 0)),
                   pl.BlockSpec((BR, HID), lambda i: (i, 0))],
        out_shape=[jax.ShapeDtypeStruct((N, HID), jnp.float32),
                   jax.ShapeDtypeStruct((N, R * HID), jnp.bfloat16),
                   jax.ShapeDtypeStruct((N, HID), jnp.float32)],
    )(p1, s1, w2r, root2, b2)


def _t3_body(p_ref, s2_ref, o1_ref, wa_ref, wb_ref, b_ref, f_ref):
    o2 = jnp.maximum(p_ref[0] + p_ref[1] + s2_ref[...], 0.0)
    f_ref[...] = (jnp.dot(o1_ref[...], wa_ref[...],
                          preferred_element_type=jnp.float32)
                  + jnp.dot(o2, wb_ref[...],
                            preferred_element_type=jnp.float32)
                  + b_ref[...])


def _tc3(p2, s2, o1, lwa, lwb, lb):
    return pl.pallas_call(
        _t3_body,
        grid=(N // BR,),
        in_specs=[pl.BlockSpec((NC, BR, HID), lambda i: (0, i, 0)),
                  pl.BlockSpec((BR, HID), lambda i: (i, 0)),
                  pl.BlockSpec((BR, HID), lambda i: (i, 0)),
                  pl.BlockSpec((HID, NCLS), lambda i: (0, 0)),
                  pl.BlockSpec((HID, NCLS), lambda i: (0, 0)),
                  pl.BlockSpec((1, NCLS), lambda i: (0, 0))],
        out_specs=pl.BlockSpec((BR, NCLS), lambda i: (i, 0)),
        out_shape=jax.ShapeDtypeStruct((N, NCLS), jnp.float32),
    )(p2, s2, o1, lwa, lwb, lb)


def kernel(x, edge_index, edge_type, weight1, root1, bias1,
           weight2, root2, bias2, lin_w, lin_b):
    src = edge_index[0]
    dst = edge_index[1]
    et = edge_type.astype(jnp.int32)
    dst2 = dst.reshape(NW, JW, CH)
    q = np.empty(R * HID, np.int32)
    for blk in range(R * HID // (2 * LANES)):
        for jj in range(LANES):
            q[blk * 2 * LANES + 2 * jj] = blk * 2 * LANES + jj
            q[blk * 2 * LANES + 2 * jj + 1] = blk * 2 * LANES + LANES + jj
    w1r = jnp.transpose(weight1, (1, 0, 2)).reshape(F_IN, R * HID)[:, q]
    w2r = jnp.transpose(weight2, (1, 0, 2)).reshape(HID, R * HID)[:, q]
    def _pack32(h):
        hi = lax.bitcast_convert_type(
            h.reshape(N, R * HID // 2, 2), jnp.int32)
        return hi.reshape(N * R, HID // 2)

    norm = _sc_norm(dst, et)
    h1, s1 = _tc1(x, w1r, root1, bias1.reshape(1, HID))
    p1 = _sc_agg(_pack32(h1), src, et, dst2, norm)
    o1, h2, s2 = _tc2(p1, s1, w2r, root2, bias2.reshape(1, HID))
    p2 = _sc_agg(_pack32(h2), src, et, dst2, norm)
    return _tc3(p2, s2, o1, lin_w[:HID], lin_w[HID:], lin_b.reshape(1, NCLS))


# speedup vs baseline: 1.9927x; 1.9927x over previous
import functools

import jax
import jax.numpy as jnp
import numpy as np
from jax import lax
from jax.experimental import pallas as pl
from jax.experimental.pallas import tpu as pltpu
from jax.experimental.pallas import tpu_sc as plsc

N = 10000
E = 320000
F_IN = 128
HID = 64
R = 8
NCLS = 32

NC = 2
NS = 16
NW = NC * NS
EPW = E // NW
CH = 80
JW = EPW // CH
ECT = E // NS
JC = ECT // CH
LANES = 16


def _norm_body(dst_hbm, et_hbm, norm_hbm,
               counts_sh, dst_v, et_v, cidx2, gcidx, cnt_v, ones_v, sem, sem2):
    cid = lax.axis_index("c")
    sid = lax.axis_index("s")
    wid = sid * NC + cid

    for k in range(CH // LANES):
        ones_v[pl.ds(k * LANES, LANES)] = jnp.ones((LANES,), jnp.float32)

    zch = (N * R) // NS

    def zero_iter(k, carry):
        cnt_v[pl.ds(k * LANES, LANES)] = jnp.zeros((LANES,), jnp.float32)
        return carry
    lax.fori_loop(0, EPW // LANES, zero_iter, None)
    pltpu.sync_copy(cnt_v.at[pl.ds(0, zch)],
                    counts_sh.at[pl.ds(sid * zch, zch)])
    plsc.subcore_barrier()

    pltpu.sync_copy(dst_hbm.at[pl.ds(sid * ECT, ECT)], dst_v)
    pltpu.sync_copy(et_hbm.at[pl.ds(sid * ECT, ECT)], et_v)

    def cidx_iter(j, carry):
        for k in range(CH // LANES):
            sl = pl.ds(j * CH + k * LANES, LANES)
            cidx2[j, pl.ds(k * LANES, LANES)] = dst_v[sl] * R + et_v[sl]
        return carry
    lax.fori_loop(0, JC, cidx_iter, None)

    kgrp = 10
    def cgrp(g, carry):
        for k in range(kgrp):
            pltpu.async_copy(ones_v, counts_sh.at[cidx2.at[g * kgrp + k]],
                             sem, add=True)
        for k in range(kgrp):
            pltpu.make_async_copy(ones_v, counts_sh.at[cidx2.at[g * kgrp + k]],
                                  sem).wait()
        return carry
    lax.fori_loop(0, JC // kgrp, cgrp, None)
    plsc.subcore_barrier()

    base = wid * EPW
    pltpu.sync_copy(dst_hbm.at[pl.ds(base, EPW)], dst_v.at[pl.ds(0, EPW)])
    pltpu.sync_copy(et_hbm.at[pl.ds(base, EPW)], et_v.at[pl.ds(0, EPW)])

    def gidx_iter(k, carry):
        sl = pl.ds(k * LANES, LANES)
        gcidx[sl] = dst_v[sl] * R + et_v[sl]
        return carry
    lax.fori_loop(0, EPW // LANES, gidx_iter, None)

    ggrp_n = 5
    def ggrp(g, carry):
        for k in range(ggrp_n):
            j = g * ggrp_n + k
            pltpu.async_copy(counts_sh.at[gcidx.at[pl.ds(j * CH, CH)]],
                             cnt_v.at[pl.ds(j * CH, CH)], sem2)
        for k in range(ggrp_n):
            j = g * ggrp_n + k
            pltpu.make_async_copy(counts_sh.at[gcidx.at[pl.ds(j * CH, CH)]],
                                  cnt_v.at[pl.ds(j * CH, CH)], sem2).wait()
        return carry
    lax.fori_loop(0, JW // ggrp_n, ggrp, None)

    def norm_iter(k, carry):
        sl = pl.ds(k * LANES, LANES)
        cnt_v[sl] = 1.0 / jnp.maximum(cnt_v[sl], 1.0)
        return carry
    lax.fori_loop(0, EPW // LANES, norm_iter, None)
    pltpu.sync_copy(cnt_v, norm_hbm.at[pl.ds(base, EPW)])


DEPTH = 4


def _agg_body(h_hbm, src_hbm, et_hbm, dst2_hbm, norm_hbm, part_hbm,
              acc_sh, src_v, et_v, dst2_v, norm_v,
              graw0, graw1, graw2, graw3, sbuf0, sbuf1, sbuf2, sbuf3,
              sg0, sg1, sg2, sg3, ss0, ss1, ss2, ss3):
    cid = lax.axis_index("c")
    sid = lax.axis_index("s")
    wid = sid * NC + cid

    NCHN = N // CH

    def zero_iter(r, carry):
        for c4 in range(HID // LANES):
            sbuf0[r, pl.ds(c4 * LANES, LANES)] = jnp.zeros((LANES,),
                                                           jnp.float32)
        return carry
    lax.fori_loop(0, CH, zero_iter, None)
    for kk in range((NCHN + NS - 1) // NS):
        cidk = sid + kk * NS
        @pl.when(cidk < NCHN)
        def _():
            pltpu.sync_copy(sbuf0, acc_sh.at[pl.ds(cidk * CH, CH)])
    plsc.subcore_barrier()

    base = wid * EPW
    pltpu.sync_copy(src_hbm.at[pl.ds(base, EPW)], src_v)
    pltpu.sync_copy(et_hbm.at[pl.ds(base, EPW)], et_v)
    pltpu.sync_copy(dst2_hbm.at[wid], dst2_v)
    pltpu.sync_copy(norm_hbm.at[pl.ds(base, EPW)], norm_v)

    def gidx_iter(k, carry):
        sl = pl.ds(k * LANES, LANES)
        src_v[sl] = src_v[sl] * R + et_v[sl]
        return carry
    lax.fori_loop(0, EPW // LANES, gidx_iter, None)

    graw = (graw0, graw1, graw2, graw3)
    sbuf = (sbuf0, sbuf1, sbuf2, sbuf3)
    sg = (sg0, sg1, sg2, sg3)
    ss = (ss0, ss1, ss2, ss3)

    def fire_g(j, b):
        pltpu.async_copy(h_hbm.at[src_v.at[pl.ds(j * CH, CH)]], graw[b], sg[b])

    def wait_g(j, b):
        pltpu.make_async_copy(h_hbm.at[src_v.at[pl.ds(j * CH, CH)]],
                              graw[b], sg[b]).wait()

    def fire_s(j, b):
        pltpu.async_copy(sbuf[b], acc_sh.at[dst2_v.at[j]], ss[b], add=True)

    def wait_s(j, b):
        pltpu.make_async_copy(sbuf[b], acc_sh.at[dst2_v.at[j]], ss[b]).wait()

    def scale(j, b):
        gb = graw[b]
        sb = sbuf[b]
        def sgrp(g, carry):
            nch = norm_v[pl.ds(j * CH + g * LANES, LANES)]
            for i in range(LANES):
                nv = nch[i]
                row = g * LANES + i
                for h2 in range(HID // (2 * LANES)):
                    w = gb[row, pl.ds(h2 * LANES, LANES)]
                    av = lax.bitcast_convert_type(w << 16, jnp.float32)
                    bv = lax.bitcast_convert_type(w & jnp.int32(-65536),
                                                  jnp.float32)
                    sb[row, pl.ds(h2 * 2 * LANES, LANES)] = av * nv
                    sb[row, pl.ds(h2 * 2 * LANES + LANES, LANES)] = bv * nv
            return carry
        lax.fori_loop(0, CH // LANES, sgrp, None)

    for jp in range(DEPTH - 1):
        fire_g(jp, jp)

    def quad(jq, carry):
        for b in range(DEPTH):
            j = jq * DEPTH + b
            @pl.when(j < JW)
            def _():
                @pl.when(j + DEPTH - 1 < JW)
                def _():
                    fire_g(j + DEPTH - 1, (b + DEPTH - 1) % DEPTH)
                wait_g(j, b)
                @pl.when(j >= DEPTH)
                def _():
                    wait_s(j - DEPTH, b)
                scale(j, b)
                fire_s(j, b)
        return carry
    lax.fori_loop(0, (JW + DEPTH - 1) // DEPTH, quad, None)
    for jt in range(JW - DEPTH, JW):
        wait_s(jt, jt % DEPTH)
    plsc.subcore_barrier()

    for kk in range((NCHN + NS - 1) // NS):
        cidk = sid + kk * NS
        @pl.when(cidk < NCHN)
        def _():
            pltpu.sync_copy(acc_sh.at[pl.ds(cidk * CH, CH)], sbuf0)
            pltpu.sync_copy(sbuf0, part_hbm.at[cid, pl.ds(cidk * CH, CH)])


def _sc_norm(dst, et):
    mesh = plsc.VectorSubcoreMesh(core_axis_name="c", subcore_axis_name="s")
    return pl.kernel(
        _norm_body,
        out_type=jax.ShapeDtypeStruct((E,), jnp.float32),
        mesh=mesh,
        compiler_params=pltpu.CompilerParams(use_tc_tiling_on_sc=False),
        scratch_types=[
            pltpu.VMEM_SHARED((N * R,), jnp.float32),
            pltpu.VMEM((ECT,), jnp.int32),
            pltpu.VMEM((ECT,), jnp.int32),
            pltpu.VMEM((JC, CH), jnp.int32),
            pltpu.VMEM((EPW,), jnp.int32),
            pltpu.VMEM((EPW,), jnp.float32),
            pltpu.VMEM((CH,), jnp.float32),
            pltpu.SemaphoreType.DMA,
            pltpu.SemaphoreType.DMA,
        ],
    )(dst, et)


def _sc_agg(h, src, et, dst2, norm):
    mesh = plsc.VectorSubcoreMesh(core_axis_name="c", subcore_axis_name="s")
    return pl.kernel(
        _agg_body,
        out_type=jax.ShapeDtypeStruct((NC, N, HID), jnp.float32),
        mesh=mesh,
        compiler_params=pltpu.CompilerParams(use_tc_tiling_on_sc=False),
        scratch_types=(
            [pltpu.VMEM_SHARED((N, HID), jnp.float32),
             pltpu.VMEM((EPW,), jnp.int32),
             pltpu.VMEM((EPW,), jnp.int32),
             pltpu.VMEM((JW, CH), jnp.int32),
             pltpu.VMEM((EPW,), jnp.float32)]
            + [pltpu.VMEM((CH, HID // 2), jnp.int32)] * DEPTH
            + [pltpu.VMEM((CH, HID), jnp.float32)] * DEPTH
            + [pltpu.SemaphoreType.DMA] * (2 * DEPTH)
        ),
    )(h, src, et, dst2, norm)


BR = 1000


def _pack_words(ha, hb):
    aw = lax.bitcast_convert_type(ha.astype(jnp.bfloat16),
                                  jnp.uint16).astype(jnp.int32)
    bw = lax.bitcast_convert_type(hb.astype(jnp.bfloat16),
                                  jnp.uint16).astype(jnp.int32)
    return aw | (bw << 16)


def _t1_body(x_ref, wa_ref, wb_ref, r_ref, b_ref, h_ref, s_ref):
    xb = x_ref[...]
    ha = jnp.dot(xb, wa_ref[...], preferred_element_type=jnp.float32)
    hb = jnp.dot(xb, wb_ref[...], preferred_element_type=jnp.float32)
    h_ref[...] = _pack_words(ha, hb)
    s_ref[...] = (jnp.dot(xb, r_ref[...], preferred_element_type=jnp.float32)
                  + b_ref[...])


RH2 = R * HID // 2


def _tc1(x, wa, wb, root1, b1):
    return pl.pallas_call(
        _t1_body,
        grid=(N // BR,),
        in_specs=[pl.BlockSpec((BR, F_IN), lambda i: (i, 0)),
                  pl.BlockSpec((F_IN, RH2), lambda i: (0, 0)),
                  pl.BlockSpec((F_IN, RH2), lambda i: (0, 0)),
                  pl.BlockSpec((F_IN, HID), lambda i: (0, 0)),
                  pl.BlockSpec((1, HID), lambda i: (0, 0))],
        out_specs=[pl.BlockSpec((BR, RH2), lambda i: (i, 0)),
                   pl.BlockSpec((BR, HID), lambda i: (i, 0))],
        out_shape=[jax.ShapeDtypeStruct((N, RH2), jnp.int32),
                   jax.ShapeDtypeStruct((N, HID), jnp.float32)],
    )(x, wa, wb, root1, b1)


def _t2_body(p_ref, s1_ref, wa_ref, wb_ref, r_ref, b_ref,
             o1_ref, h2_ref, s2_ref):
    o1 = jnp.maximum(p_ref[0] + p_ref[1] + s1_ref[...], 0.0)
    o1_ref[...] = o1
    ha = jnp.dot(o1, wa_ref[...], preferred_element_type=jnp.float32)
    hb = jnp.dot(o1, wb_ref[...], preferred_element_type=jnp.float32)
    h2_ref[...] = _pack_words(ha, hb)
    s2_ref[...] = (jnp.dot(o1, r_ref[...], preferred_element_type=jnp.float32)
                   + b_ref[...])


def _tc2(p1, s1, wa, wb, root2, b2):
    return pl.pallas_call(
        _t2_body,
        grid=(N // BR,),
        in_specs=[pl.BlockSpec((NC, BR, HID), lambda i: (0, i, 0)),
                  pl.BlockSpec((BR, HID), lambda i: (i, 0)),
                  pl.BlockSpec((HID, RH2), lambda i: (0, 0)),
                  pl.BlockSpec((HID, RH2), lambda i: (0, 0)),
                  pl.BlockSpec((HID, HID), lambda i: (0, 0)),
                  pl.BlockSpec((1, HID), lambda i: (0, 0))],
        out_specs=[pl.BlockSpec((BR, HID), lambda i: (i, 0)),
                   pl.BlockSpec((BR, RH2), lambda i: (i, 0)),
                   pl.BlockSpec((BR, HID), lambda i: (i, 0))],
        out_shape=[jax.ShapeDtypeStruct((N, HID), jnp.float32),
                   jax.ShapeDtypeStruct((N, RH2), jnp.int32),
                   jax.ShapeDtypeStruct((N, HID), jnp.float32)],
    )(p1, s1, wa, wb, root2, b2)


def _t3_body(p_ref, s2_ref, o1_ref, wa_ref, wb_ref, b_ref, f_ref):
    o2 = jnp.maximum(p_ref[0] + p_ref[1] + s2_ref[...], 0.0)
    f_ref[...] = (jnp.dot(o1_ref[...], wa_ref[...],
                          preferred_element_type=jnp.float32)
                  + jnp.dot(o2, wb_ref[...],
                            preferred_element_type=jnp.float32)
                  + b_ref[...])


def _tc3(p2, s2, o1, lwa, lwb, lb):
    return pl.pallas_call(
        _t3_body,
        grid=(N // BR,),
        in_specs=[pl.BlockSpec((NC, BR, HID), lambda i: (0, i, 0)),
                  pl.BlockSpec((BR, HID), lambda i: (i, 0)),
                  pl.BlockSpec((BR, HID), lambda i: (i, 0)),
                  pl.BlockSpec((HID, NCLS), lambda i: (0, 0)),
                  pl.BlockSpec((HID, NCLS), lambda i: (0, 0)),
                  pl.BlockSpec((1, NCLS), lambda i: (0, 0))],
        out_specs=pl.BlockSpec((BR, NCLS), lambda i: (i, 0)),
        out_shape=jax.ShapeDtypeStruct((N, NCLS), jnp.float32),
    )(p2, s2, o1, lwa, lwb, lb)


def kernel(x, edge_index, edge_type, weight1, root1, bias1,
           weight2, root2, bias2, lin_w, lin_b):
    src = edge_index[0]
    dst = edge_index[1]
    et = edge_type.astype(jnp.int32)
    dst2 = dst.reshape(NW, JW, CH)
    qa = np.empty(RH2, np.int32)
    for rel in range(R):
        for h2 in range(HID // (2 * LANES)):
            for jj in range(LANES):
                qa[rel * (HID // 2) + h2 * LANES + jj] = (
                    rel * HID + h2 * 2 * LANES + jj)
    qb = qa + LANES
    w1r = jnp.transpose(weight1, (1, 0, 2)).reshape(F_IN, R * HID)
    w2r = jnp.transpose(weight2, (1, 0, 2)).reshape(HID, R * HID)
    w1a, w1b = w1r[:, qa], w1r[:, qb]
    w2a, w2b = w2r[:, qa], w2r[:, qb]
    norm = _sc_norm(dst, et)
    h1, s1 = _tc1(x, w1a, w1b, root1, bias1.reshape(1, HID))
    p1 = _sc_agg(h1.reshape(N * R, HID // 2), src, et, dst2, norm)
    o1, h2, s2 = _tc2(p1, s1, w2a, w2b, root2, bias2.reshape(1, HID))
    p2 = _sc_agg(h2.reshape(N * R, HID // 2), src, et, dst2, norm)
    return _tc3(p2, s2, o1, lin_w[:HID], lin_w[HID:], lin_b.reshape(1, NCLS))


# speedup vs baseline: 2.7910x; 1.4006x over previous
import functools

import jax
import jax.numpy as jnp
import numpy as np
from jax import lax
from jax.experimental import pallas as pl
from jax.experimental.pallas import tpu as pltpu
from jax.experimental.pallas import tpu_sc as plsc

N = 10000
E = 320000
F_IN = 128
HID = 64
R = 8
NCLS = 32

NC = 2
NS = 16
NW = NC * NS
EPW = E // NW
CH = 80
JW = EPW // CH
ECT = E // NS
JC = ECT // CH
LANES = 16
DEPTH = 4


def _norm_body(dst_hbm, et_hbm, norm_hbm,
               counts_sh, dst_v, et_v, cidx2, gcidx, cnt_v, ones_v, sem, sem2):
    cid = lax.axis_index("c")
    sid = lax.axis_index("s")
    wid = sid * NC + cid

    for k in range(CH // LANES):
        ones_v[pl.ds(k * LANES, LANES)] = jnp.ones((LANES,), jnp.float32)

    zch = (N * R) // NS

    def zero_iter(k, carry):
        cnt_v[pl.ds(k * LANES, LANES)] = jnp.zeros((LANES,), jnp.float32)
        return carry
    lax.fori_loop(0, EPW // LANES, zero_iter, None)
    pltpu.sync_copy(cnt_v.at[pl.ds(0, zch)],
                    counts_sh.at[pl.ds(sid * zch, zch)])
    plsc.subcore_barrier()

    pltpu.sync_copy(dst_hbm.at[pl.ds(sid * ECT, ECT)], dst_v)
    pltpu.sync_copy(et_hbm.at[pl.ds(sid * ECT, ECT)], et_v)

    def cidx_iter(j, carry):
        for k in range(CH // LANES):
            sl = pl.ds(j * CH + k * LANES, LANES)
            cidx2[j, pl.ds(k * LANES, LANES)] = dst_v[sl] * R + et_v[sl]
        return carry
    lax.fori_loop(0, JC, cidx_iter, None)

    kgrp = 10
    def cgrp(g, carry):
        for k in range(kgrp):
            pltpu.async_copy(ones_v, counts_sh.at[cidx2.at[g * kgrp + k]],
                             sem, add=True)
        for k in range(kgrp):
            pltpu.make_async_copy(ones_v, counts_sh.at[cidx2.at[g * kgrp + k]],
                                  sem).wait()
        return carry
    lax.fori_loop(0, JC // kgrp, cgrp, None)
    plsc.subcore_barrier()

    base = wid * EPW
    pltpu.sync_copy(dst_hbm.at[pl.ds(base, EPW)], dst_v.at[pl.ds(0, EPW)])
    pltpu.sync_copy(et_hbm.at[pl.ds(base, EPW)], et_v.at[pl.ds(0, EPW)])

    def gidx_iter(k, carry):
        sl = pl.ds(k * LANES, LANES)
        gcidx[sl] = dst_v[sl] * R + et_v[sl]
        return carry
    lax.fori_loop(0, EPW // LANES, gidx_iter, None)

    ggrp_n = 5
    def ggrp(g, carry):
        for k in range(ggrp_n):
            j = g * ggrp_n + k
            pltpu.async_copy(counts_sh.at[gcidx.at[pl.ds(j * CH, CH)]],
                             cnt_v.at[pl.ds(j * CH, CH)], sem2)
        for k in range(ggrp_n):
            j = g * ggrp_n + k
            pltpu.make_async_copy(counts_sh.at[gcidx.at[pl.ds(j * CH, CH)]],
                                  cnt_v.at[pl.ds(j * CH, CH)], sem2).wait()
        return carry
    lax.fori_loop(0, JW // ggrp_n, ggrp, None)

    def norm_iter(k, carry):
        sl = pl.ds(k * LANES, LANES)
        cnt_v[sl] = 1.0 / jnp.maximum(cnt_v[sl], 1.0)
        return carry
    lax.fori_loop(0, EPW // LANES, norm_iter, None)
    pltpu.sync_copy(cnt_v, norm_hbm.at[pl.ds(base, EPW)])


def _agg_body(h_hbm, src_hbm, et_hbm, dst2_hbm, norm_hbm, part_hbm,
              acc_sh, src_v, et_v, dst2_v, norm_v,
              graw0, graw1, graw2, graw3, sbuf0, sbuf1, sbuf2, sbuf3,
              sg0, sg1, sg2, sg3, ss0, ss1, ss2, ss3):
    cid = lax.axis_index("c")
    sid = lax.axis_index("s")
    wid = sid * NC + cid

    NCHN = N // CH

    def zero_iter(r, carry):
        for c4 in range(HID // LANES):
            sbuf0[r, pl.ds(c4 * LANES, LANES)] = jnp.zeros((LANES,),
                                                           jnp.float32)
        return carry
    lax.fori_loop(0, CH, zero_iter, None)
    for kk in range((NCHN + NS - 1) // NS):
        cidk = sid + kk * NS
        @pl.when(cidk < NCHN)
        def _():
            pltpu.sync_copy(sbuf0, acc_sh.at[pl.ds(cidk * CH, CH)])
    plsc.subcore_barrier()

    base = wid * EPW
    pltpu.sync_copy(src_hbm.at[pl.ds(base, EPW)], src_v)
    pltpu.sync_copy(et_hbm.at[pl.ds(base, EPW)], et_v)
    pltpu.sync_copy(dst2_hbm.at[wid], dst2_v)
    pltpu.sync_copy(norm_hbm.at[pl.ds(base, EPW)], norm_v)

    def gidx_iter(k, carry):
        sl = pl.ds(k * LANES, LANES)
        src_v[sl] = src_v[sl] * R + et_v[sl]
        return carry
    lax.fori_loop(0, EPW // LANES, gidx_iter, None)

    graw = (graw0, graw1, graw2, graw3)
    sbuf = (sbuf0, sbuf1, sbuf2, sbuf3)
    sg = (sg0, sg1, sg2, sg3)
    ss = (ss0, ss1, ss2, ss3)

    def fire_g(j, b):
        pltpu.async_copy(h_hbm.at[src_v.at[pl.ds(j * CH, CH)]], graw[b], sg[b])

    def wait_g(j, b):
        pltpu.make_async_copy(h_hbm.at[src_v.at[pl.ds(j * CH, CH)]],
                              graw[b], sg[b]).wait()

    def fire_s(j, b):
        pltpu.async_copy(sbuf[b], acc_sh.at[dst2_v.at[j]], ss[b], add=True)

    def wait_s(j, b):
        pltpu.make_async_copy(sbuf[b], acc_sh.at[dst2_v.at[j]], ss[b]).wait()

    def scale(j, b):
        gb = graw[b]
        sb = sbuf[b]
        def sgrp(g, carry):
            nch = norm_v[pl.ds(j * CH + g * LANES, LANES)]
            for i in range(LANES):
                nv = nch[i]
                row = g * LANES + i
                for c4 in range(HID // LANES):
                    sl = pl.ds(c4 * LANES, LANES)
                    sb[row, sl] = gb[row, sl] * nv
            return carry
        lax.fori_loop(0, CH // LANES, sgrp, None)

    for jp in range(DEPTH - 1):
        fire_g(jp, jp)

    def quad(jq, carry):
        for b in range(DEPTH):
            j = jq * DEPTH + b
            @pl.when(j < JW)
            def _():
                @pl.when(j + DEPTH - 1 < JW)
                def _():
                    fire_g(j + DEPTH - 1, (b + DEPTH - 1) % DEPTH)
                wait_g(j, b)
                @pl.when(j >= DEPTH)
                def _():
                    wait_s(j - DEPTH, b)
                scale(j, b)
                fire_s(j, b)
        return carry
    lax.fori_loop(0, (JW + DEPTH - 1) // DEPTH, quad, None)
    for jt in range(JW - DEPTH, JW):
        wait_s(jt, jt % DEPTH)
    plsc.subcore_barrier()

    for kk in range((NCHN + NS - 1) // NS):
        cidk = sid + kk * NS
        @pl.when(cidk < NCHN)
        def _():
            pltpu.sync_copy(acc_sh.at[pl.ds(cidk * CH, CH)], sbuf0)
            pltpu.sync_copy(sbuf0, part_hbm.at[cid, pl.ds(cidk * CH, CH)])


def _sc_norm(dst, et):
    mesh = plsc.VectorSubcoreMesh(core_axis_name="c", subcore_axis_name="s")
    return pl.kernel(
        _norm_body,
        out_type=jax.ShapeDtypeStruct((E,), jnp.float32),
        mesh=mesh,
        compiler_params=pltpu.CompilerParams(use_tc_tiling_on_sc=False),
        scratch_types=[
            pltpu.VMEM_SHARED((N * R,), jnp.float32),
            pltpu.VMEM((ECT,), jnp.int32),
            pltpu.VMEM((ECT,), jnp.int32),
            pltpu.VMEM((JC, CH), jnp.int32),
            pltpu.VMEM((EPW,), jnp.int32),
            pltpu.VMEM((EPW,), jnp.float32),
            pltpu.VMEM((CH,), jnp.float32),
            pltpu.SemaphoreType.DMA,
            pltpu.SemaphoreType.DMA,
        ],
    )(dst, et)


def _sc_agg(h, src, et, dst2, norm):
    mesh = plsc.VectorSubcoreMesh(core_axis_name="c", subcore_axis_name="s")
    return pl.kernel(
        _agg_body,
        out_type=jax.ShapeDtypeStruct((NC, N, HID), jnp.float32),
        mesh=mesh,
        compiler_params=pltpu.CompilerParams(use_tc_tiling_on_sc=False),
        scratch_types=(
            [pltpu.VMEM_SHARED((N, HID), jnp.float32),
             pltpu.VMEM((EPW,), jnp.int32),
             pltpu.VMEM((EPW,), jnp.int32),
             pltpu.VMEM((JW, CH), jnp.int32),
             pltpu.VMEM((EPW,), jnp.float32)]
            + [pltpu.VMEM((CH, HID), jnp.float32)] * (2 * DEPTH)
            + [pltpu.SemaphoreType.DMA] * (2 * DEPTH)
        ),
    )(h, src, et, dst2, norm)


BR = 1000


def _t1_body(x_ref, w_ref, r_ref, b_ref, h_ref, s_ref):
    xb = x_ref[...]
    h_ref[...] = jnp.dot(xb, w_ref[...], preferred_element_type=jnp.float32)
    s_ref[...] = (jnp.dot(xb, r_ref[...], preferred_element_type=jnp.float32)
                  + b_ref[...])


def _tc1(x, w1r, root1, b1):
    return pl.pallas_call(
        _t1_body,
        grid=(N // BR,),
        in_specs=[pl.BlockSpec((BR, F_IN), lambda i: (i, 0)),
                  pl.BlockSpec((F_IN, R * HID), lambda i: (0, 0)),
                  pl.BlockSpec((F_IN, HID), lambda i: (0, 0)),
                  pl.BlockSpec((1, HID), lambda i: (0, 0))],
        out_specs=[pl.BlockSpec((BR, R * HID), lambda i: (i, 0)),
                   pl.BlockSpec((BR, HID), lambda i: (i, 0))],
        out_shape=[jax.ShapeDtypeStruct((N, R * HID), jnp.float32),
                   jax.ShapeDtypeStruct((N, HID), jnp.float32)],
    )(x, w1r, root1, b1)


def _t2_body(p_ref, s1_ref, w_ref, r_ref, b_ref, o1_ref, h2_ref, s2_ref):
    o1 = jnp.maximum(p_ref[0] + p_ref[1] + s1_ref[...], 0.0)
    o1_ref[...] = o1
    h2_ref[...] = jnp.dot(o1, w_ref[...], preferred_element_type=jnp.float32)
    s2_ref[...] = (jnp.dot(o1, r_ref[...], preferred_element_type=jnp.float32)
                   + b_ref[...])


def _tc2(p1, s1, w2r, root2, b2):
    return pl.pallas_call(
        _t2_body,
        grid=(N // BR,),
        in_specs=[pl.BlockSpec((NC, BR, HID), lambda i: (0, i, 0)),
                  pl.BlockSpec((BR, HID), lambda i: (i, 0)),
                  pl.BlockSpec((HID, R * HID), lambda i: (0, 0)),
                  pl.BlockSpec((HID, HID), lambda i: (0, 0)),
                  pl.BlockSpec((1, HID), lambda i: (0, 0))],
        out_specs=[pl.BlockSpec((BR, HID), lambda i: (i, 0)),
                   pl.BlockSpec((BR, R * HID), lambda i: (i, 0)),
                   pl.BlockSpec((BR, HID), lambda i: (i, 0))],
        out_shape=[jax.ShapeDtypeStruct((N, HID), jnp.float32),
                   jax.ShapeDtypeStruct((N, R * HID), jnp.float32),
                   jax.ShapeDtypeStruct((N, HID), jnp.float32)],
    )(p1, s1, w2r, root2, b2)


def _t3_body(p_ref, s2_ref, o1_ref, wa_ref, wb_ref, b_ref, f_ref):
    o2 = jnp.maximum(p_ref[0] + p_ref[1] + s2_ref[...], 0.0)
    f_ref[...] = (jnp.dot(o1_ref[...], wa_ref[...],
                          preferred_element_type=jnp.float32)
                  + jnp.dot(o2, wb_ref[...],
                            preferred_element_type=jnp.float32)
                  + b_ref[...])


def _tc3(p2, s2, o1, lwa, lwb, lb):
    return pl.pallas_call(
        _t3_body,
        grid=(N // BR,),
        in_specs=[pl.BlockSpec((NC, BR, HID), lambda i: (0, i, 0)),
                  pl.BlockSpec((BR, HID), lambda i: (i, 0)),
                  pl.BlockSpec((BR, HID), lambda i: (i, 0)),
                  pl.BlockSpec((HID, NCLS), lambda i: (0, 0)),
                  pl.BlockSpec((HID, NCLS), lambda i: (0, 0)),
                  pl.BlockSpec((1, NCLS), lambda i: (0, 0))],
        out_specs=pl.BlockSpec((BR, NCLS), lambda i: (i, 0)),
        out_shape=jax.ShapeDtypeStruct((N, NCLS), jnp.float32),
    )(p2, s2, o1, lwa, lwb, lb)


def kernel(x, edge_index, edge_type, weight1, root1, bias1,
           weight2, root2, bias2, lin_w, lin_b):
    src = edge_index[0]
    dst = edge_index[1]
    et = edge_type.astype(jnp.int32)
    dst2 = dst.reshape(NW, JW, CH)
    w1r = jnp.transpose(weight1, (1, 0, 2)).reshape(F_IN, R * HID)
    w2r = jnp.transpose(weight2, (1, 0, 2)).reshape(HID, R * HID)

    norm = _sc_norm(dst, et)
    h1, s1 = _tc1(x, w1r, root1, bias1.reshape(1, HID))
    p1 = _sc_agg(h1.reshape(N * R, HID), src, et, dst2, norm)
    o1, h2, s2 = _tc2(p1, s1, w2r, root2, bias2.reshape(1, HID))
    p2 = _sc_agg(h2.reshape(N * R, HID), src, et, dst2, norm)
    return _tc3(p2, s2, o1, lin_w[:HID], lin_w[HID:], lin_b.reshape(1, NCLS))
